# Initial kernel scaffold; baseline (speedup 1.0000x reference)
import jax
import jax.numpy as jnp
from jax.experimental import pallas as pl


def kernel(timedelta, table):
    raise NotImplementedError("write your pallas kernel here")


# speedup vs baseline: 4.0202x; 4.0202x over previous
import jax
import jax.numpy as jnp
from jax.experimental import pallas as pl
from jax.experimental.pallas import tpu as pltpu
from jax.experimental.pallas import tpu_sc as plsc

_WINDOW = 256


def kernel(timedelta, table):
    B, T = timedelta.shape
    V, D = table.shape
    N = B * T
    M = N // 2

    idx = timedelta.reshape(-1).astype(jnp.int32)
    pair_idx = (idx[0::2] * V + idx[1::2]).reshape(1, M)
    pair_table = jnp.concatenate(
        [
            jnp.broadcast_to(table[:, None, :], (V, V, D)),
            jnp.broadcast_to(table[None, :, :], (V, V, D)),
        ],
        axis=-1,
    ).reshape(V * V, 2 * D)

    mesh = plsc.VectorSubcoreMesh(core_axis_name="core", subcore_axis_name="subcore")

    @pl.kernel(out_type=jax.ShapeDtypeStruct((M, 2 * D), table.dtype), mesh=mesh)
    def _lookup(table_hbm, i_hbm, o_hbm):
        def body(i_vmem, o_vmem):
            pltpu.sync_copy(table_hbm.at[i_vmem.at[0]], o_vmem)

        pltpu.emit_pipeline(
            body,
            grid=(M // _WINDOW,),
            in_specs=[pl.BlockSpec((1, _WINDOW), index_map=lambda i: (0, i))],
            out_specs=[pl.BlockSpec((_WINDOW, 2 * D), index_map=lambda i: (i, 0))],
            core_axis_name=("core", "subcore"),
            dimension_semantics=(pltpu.PARALLEL,),
        )(i_hbm, o_hbm)

    out = _lookup(pair_table, pair_idx)
    return out.reshape(B, T, D)
